# Initial kernel scaffold; baseline (speedup 1.0000x reference)
import jax
import jax.numpy as jnp
from jax.experimental import pallas as pl


def kernel(x, edge_index, edge_label_index, W1l, b1, W1r, W2l, b2, W2r, Wd1, bd1, Wd2, bd2):
    raise NotImplementedError("write your pallas kernel here")


# speedup vs baseline: 4.4587x; 4.4587x over previous
import dataclasses
import functools

import jax
import jax.numpy as jnp
from jax import lax
from jax.experimental import pallas as pl
from jax.experimental.pallas import tpu as pltpu
from jax.experimental.pallas import tpu_sc as plsc

_N = 10000
_NP = 10240
_H = 128
_E = 320000
_EL = 100000
_EB = 128
_NBAT = _E // _EB
_NW = 32
_RPS = _NP // 16
_ELP = 102400
_NBL = _ELP // _EB

_f32 = jnp.float32


def _mesh():
    return plsc.VectorSubcoreMesh(core_axis_name="c", subcore_axis_name="s")


def _segsum(y, src1d, dst1d):
    @functools.partial(
        pl.kernel,
        out_type=jax.ShapeDtypeStruct((2 * _NP, _H), _f32),
        mesh=_mesh(),
        scratch_types=[
            pltpu.VMEM((_EB,), jnp.int32),
            pltpu.VMEM((_EB,), jnp.int32),
            pltpu.VMEM((_EB, _H), _f32),
            pltpu.VMEM_SHARED((_NP, _H), _f32),
        ],
    )
    def k(y_hbm, src_hbm, dst_hbm, s_hbm, srcv, dstv, rows, acc):
        c = lax.axis_index("c")
        s = lax.axis_index("s")
        w = s * 2 + c
        r0 = pl.multiple_of(s * _RPS, _RPS)

        @pl.loop(0, _EB)
        def _(i):
            @pl.loop(0, _H // 16)
            def _(j):
                rows[i, pl.ds(j * 16, 16)] = jnp.zeros((16,), _f32)

        @pl.loop(0, _RPS // _EB)
        def _(i):
            pltpu.sync_copy(rows, acc.at[pl.ds(r0 + i * _EB, _EB)])

        plsc.subcore_barrier()

        @pl.loop(0, (_NBAT + _NW - 1) // _NW)
        def _(t):
            j = w + t * _NW

            @pl.when(j < _NBAT)
            def _():
                e0 = pl.multiple_of(j * _EB, _EB)
                pltpu.sync_copy(src_hbm.at[pl.ds(e0, _EB)], srcv)
                pltpu.sync_copy(dst_hbm.at[pl.ds(e0, _EB)], dstv)
                pltpu.sync_copy(y_hbm.at[srcv], rows)
                pltpu.sync_copy(rows, acc.at[dstv], add=True)

        plsc.subcore_barrier()
        o0 = pl.multiple_of(c * _NP + r0, _RPS)
        pltpu.sync_copy(acc.at[pl.ds(r0, _RPS)], s_hbm.at[pl.ds(o0, _RPS)])

    return k(y, src1d, dst1d)


_HR = _NP // _H
_HRT = _HR // 16


def _deg_hist(dst1d):
    @functools.partial(
        pl.kernel,
        out_type=jax.ShapeDtypeStruct((2 * _NP * 16,), _f32),
        mesh=_mesh(),
        compiler_params=dataclasses.replace(
            pltpu.CompilerParams(), needs_layout_passes=False),
        scratch_types=[
            pltpu.VMEM((_EB,), jnp.int32),
            pltpu.VMEM((_HR, _H), _f32),
            pltpu.VMEM((_HR,), jnp.int32),
            pltpu.VMEM((_HRT, _H), _f32),
            pltpu.VMEM((_RPS * 16,), _f32),
            pltpu.VMEM_SHARED((_HR, _H), _f32),
        ],
    )
    def k(dst_hbm, d_hbm, dstv, hist, idv, dbuf, bbuf, sdeg):
        c = lax.axis_index("c")
        s = lax.axis_index("s")
        w = s * 2 + c

        @pl.loop(0, _HR)
        def _(i):
            @pl.loop(0, _H // 16)
            def _(j):
                hist[i, pl.ds(j * 16, 16)] = jnp.zeros((16,), _f32)

        @pl.loop(0, _HR // 16)
        def _(i):
            idv[pl.ds(i * 16, 16)] = lax.iota(jnp.int32, 16) + i * 16

        @pl.when(s == 0)
        def _():
            pltpu.sync_copy(hist, sdeg)

        plsc.subcore_barrier()

        ones16 = jnp.ones((16,), _f32)

        @pl.loop(0, (_NBAT + _NW - 1) // _NW)
        def _(t):
            j = w + t * _NW

            @pl.when(j < _NBAT)
            def _():
                e0 = pl.multiple_of(j * _EB, _EB)
                pltpu.sync_copy(dst_hbm.at[pl.ds(e0, _EB)], dstv)

                @pl.loop(0, _EB // 16)
                def _(kk):
                    d16 = dstv[pl.ds(kk * 16, 16)]
                    r = lax.shift_right_logical(d16, 7)
                    cl = lax.bitwise_and(d16, 127)
                    plsc.addupdate_scatter(hist, [r, cl], ones16)

        pltpu.sync_copy(hist, sdeg.at[idv], add=True)
        plsc.subcore_barrier()

        pltpu.sync_copy(sdeg.at[pl.ds(s * _HRT, _HRT)], dbuf)

        @pl.loop(0, _HRT)
        def _(r):
            @pl.loop(0, _H // 16)
            def _(lg):
                v = dbuf[r, pl.ds(lg * 16, 16)]
                base = (r * _H + lg * 16) * 16
                for ll in range(16):
                    bbuf[pl.ds(base + ll * 16, 16)] = jnp.full((16,), v[ll], _f32)

        o0 = pl.multiple_of((c * _NP + s * _RPS) * 16, 8)
        pltpu.sync_copy(bbuf, d_hbm.at[pl.ds(o0, _RPS * 16)])

    return k(dst1d)


def _edge_gather(u, v, e0, e1):
    @functools.partial(
        pl.kernel,
        out_type=(jax.ShapeDtypeStruct((_ELP, _H), _f32),
                  jax.ShapeDtypeStruct((_ELP, _H), _f32)),
        mesh=_mesh(),
        scratch_types=[
            pltpu.VMEM((_EB,), jnp.int32),
            pltpu.VMEM((_EB,), jnp.int32),
            pltpu.VMEM((_EB, _H), _f32),
            pltpu.VMEM((_EB, _H), _f32),
        ],
    )
    def k(u_hbm, v_hbm, e0_hbm, e1_hbm, ug_hbm, vg_hbm, i0, i1, ru, rv):
        c = lax.axis_index("c")
        s = lax.axis_index("s")
        w = s * 2 + c

        @pl.loop(0, _NBL // _NW)
        def _(t):
            j = w + t * _NW
            b0 = pl.multiple_of(j * _EB, _EB)
            pltpu.sync_copy(e0_hbm.at[pl.ds(b0, _EB)], i0)
            pltpu.sync_copy(e1_hbm.at[pl.ds(b0, _EB)], i1)
            pltpu.sync_copy(u_hbm.at[i0], ru)
            pltpu.sync_copy(ru, ug_hbm.at[pl.ds(b0, _EB)])
            pltpu.sync_copy(v_hbm.at[i1], rv)
            pltpu.sync_copy(rv, vg_hbm.at[pl.ds(b0, _EB)])

    return k(u, v, e0, e1)


def _tc_pre(x, W1l, W1r, b1):
    R = 1024

    def body(x_ref, wl_ref, wr_ref, b_ref, y_ref, xr_ref):
        xx = x_ref[...]
        y_ref[...] = jnp.dot(xx, wl_ref[...], preferred_element_type=_f32)
        xr_ref[...] = (jnp.dot(xx, wr_ref[...], preferred_element_type=_f32)
                       + b_ref[...])

    return pl.pallas_call(
        body,
        grid=(_NP // R,),
        in_specs=[pl.BlockSpec((R, _H), lambda i: (i, 0)),
                  pl.BlockSpec((_H, _H), lambda i: (0, 0)),
                  pl.BlockSpec((_H, _H), lambda i: (0, 0)),
                  pl.BlockSpec((1, _H), lambda i: (0, 0))],
        out_specs=[pl.BlockSpec((R, _H), lambda i: (i, 0)),
                   pl.BlockSpec((R, _H), lambda i: (i, 0))],
        out_shape=[jax.ShapeDtypeStruct((_NP, _H), _f32)] * 2,
    )(x, W1l, W1r, b1.reshape(1, _H))


def _tc_mid(s1p, degp, xr1b, W2l, W2r, b2):
    R = 1024

    def body(sp, dp, xr, wl, wr, b, y2, hr):
        ssum = sp[0] + sp[1]
        dg = dp[0, :, 0:1] + dp[1, :, 0:1]
        inv = 1.0 / jnp.maximum(dg, 1.0)
        h = jnp.maximum(ssum * inv + xr[...], 0.0)
        y2[...] = jnp.dot(h, wl[...], preferred_element_type=_f32)
        hr[...] = jnp.dot(h, wr[...], preferred_element_type=_f32) + b[...]

    return pl.pallas_call(
        body,
        grid=(_NP // R,),
        in_specs=[pl.BlockSpec((2, R, _H), lambda i: (0, i, 0)),
                  pl.BlockSpec((2, R, 16), lambda i: (0, i, 0)),
                  pl.BlockSpec((R, _H), lambda i: (i, 0)),
                  pl.BlockSpec((_H, _H), lambda i: (0, 0)),
                  pl.BlockSpec((_H, _H), lambda i: (0, 0)),
                  pl.BlockSpec((1, _H), lambda i: (0, 0))],
        out_specs=[pl.BlockSpec((R, _H), lambda i: (i, 0)),
                   pl.BlockSpec((R, _H), lambda i: (i, 0))],
        out_shape=[jax.ShapeDtypeStruct((_NP, _H), _f32)] * 2,
    )(s1p, degp, xr1b, W2l, W2r, b2.reshape(1, _H))


def _tc_z(s2p, degp, hr2b, Wd1, bd1):
    R = 1024

    def body(sp, dp, hr, wd, b, u, v):
        ssum = sp[0] + sp[1]
        dg = dp[0, :, 0:1] + dp[1, :, 0:1]
        inv = 1.0 / jnp.maximum(dg, 1.0)
        z = ssum * inv + hr[...]
        wa = wd[0:_H, :]
        wb = wd[_H:2 * _H, :]
        u[...] = jnp.dot(z, wa, preferred_element_type=_f32) + b[...]
        v[...] = jnp.dot(z, wb, preferred_element_type=_f32)

    return pl.pallas_call(
        body,
        grid=(_NP // R,),
        in_specs=[pl.BlockSpec((2, R, _H), lambda i: (0, i, 0)),
                  pl.BlockSpec((2, R, 16), lambda i: (0, i, 0)),
                  pl.BlockSpec((R, _H), lambda i: (i, 0)),
                  pl.BlockSpec((2 * _H, _H), lambda i: (0, 0)),
                  pl.BlockSpec((1, _H), lambda i: (0, 0))],
        out_specs=[pl.BlockSpec((R, _H), lambda i: (i, 0)),
                   pl.BlockSpec((R, _H), lambda i: (i, 0))],
        out_shape=[jax.ShapeDtypeStruct((_NP, _H), _f32)] * 2,
    )(s2p, degp, hr2b, Wd1, bd1.reshape(1, _H))


def _tc_dec(ug, vg, wd2r, bd2):
    R = 1024

    def body(a, b, w, bb, o):
        t = jnp.maximum(a[...] + b[...], 0.0)
        val = jnp.sum(t * w[...], axis=1) + bb[0, 0]
        o[...] = val.reshape(R // _H, _H)

    return pl.pallas_call(
        body,
        grid=(_ELP // R,),
        in_specs=[pl.BlockSpec((R, _H), lambda i: (i, 0)),
                  pl.BlockSpec((R, _H), lambda i: (i, 0)),
                  pl.BlockSpec((1, _H), lambda i: (0, 0)),
                  pl.BlockSpec((1, 1), lambda i: (0, 0))],
        out_specs=pl.BlockSpec((R // _H, _H), lambda i: (i, 0)),
        out_shape=jax.ShapeDtypeStruct((_ELP // _H, _H), _f32),
    )(ug, vg, wd2r, bd2)


def kernel(x, edge_index, edge_label_index,
           W1l, b1, W1r, W2l, b2, W2r, Wd1, bd1, Wd2, bd2):
    xp = jnp.pad(x, ((0, _NP - _N), (0, 0)))
    src1d = edge_index[0]
    dst1d = edge_index[1]

    y1, xr1b = _tc_pre(xp, W1l, W1r, b1)
    degf = _deg_hist(dst1d)
    s1f = _segsum(y1, src1d, dst1d)
    s1p = s1f.reshape(2, _NP, _H)
    degp = degf.reshape(2, _NP, 16)

    y2, hr2b = _tc_mid(s1p, degp, xr1b, W2l, W2r, b2)
    s2f = _segsum(y2, src1d, dst1d)
    s2p = s2f.reshape(2, _NP, _H)

    u, v = _tc_z(s2p, degp, hr2b, Wd1, bd1)

    pad = _ELP - _EL
    e0 = jnp.pad(edge_label_index[0], (0, pad))
    e1 = jnp.pad(edge_label_index[1], (0, pad))
    ug, vg = _edge_gather(u, v, e0, e1)

    out2d = _tc_dec(ug, vg, Wd2.reshape(1, _H), bd2.reshape(1, 1))
    return out2d.reshape(-1)[:_EL]


# speedup vs baseline: 4.7898x; 1.0743x over previous
import dataclasses
import functools

import jax
import jax.numpy as jnp
from jax import lax
from jax.experimental import pallas as pl
from jax.experimental.pallas import tpu as pltpu
from jax.experimental.pallas import tpu_sc as plsc

_N = 10000
_NP = 10240
_H = 128
_E = 320000
_EL = 100000
_EB = 128
_NBAT = _E // _EB
_NW = 32
_RPS = _NP // 16
_ELP = 102400
_NBL = _ELP // _EB

_f32 = jnp.float32


def _mesh():
    return plsc.VectorSubcoreMesh(core_axis_name="c", subcore_axis_name="s")


def _segsum(y, src1d, dst1d):
    @functools.partial(
        pl.kernel,
        out_type=jax.ShapeDtypeStruct((2 * _NP, _H), _f32),
        mesh=_mesh(),
        scratch_types=[
            pltpu.VMEM((_EB,), jnp.int32),
            pltpu.VMEM((_EB,), jnp.int32),
            pltpu.VMEM((_EB, _H), _f32),
            pltpu.VMEM_SHARED((_NP, _H), _f32),
        ],
    )
    def k(y_hbm, src_hbm, dst_hbm, s_hbm, srcv, dstv, rows, acc):
        c = lax.axis_index("c")
        s = lax.axis_index("s")
        w = s * 2 + c
        r0 = pl.multiple_of(s * _RPS, _RPS)

        @pl.loop(0, _EB)
        def _(i):
            @pl.loop(0, _H // 16)
            def _(j):
                rows[i, pl.ds(j * 16, 16)] = jnp.zeros((16,), _f32)

        @pl.loop(0, _RPS // _EB)
        def _(i):
            pltpu.sync_copy(rows, acc.at[pl.ds(r0 + i * _EB, _EB)])

        plsc.subcore_barrier()

        @pl.loop(0, (_NBAT + _NW - 1) // _NW)
        def _(t):
            j = w + t * _NW

            @pl.when(j < _NBAT)
            def _():
                e0 = pl.multiple_of(j * _EB, _EB)
                pltpu.sync_copy(src_hbm.at[pl.ds(e0, _EB)], srcv)
                pltpu.sync_copy(dst_hbm.at[pl.ds(e0, _EB)], dstv)
                pltpu.sync_copy(y_hbm.at[srcv], rows)
                pltpu.sync_copy(rows, acc.at[dstv], add=True)

        plsc.subcore_barrier()
        o0 = pl.multiple_of(c * _NP + r0, _RPS)
        pltpu.sync_copy(acc.at[pl.ds(r0, _RPS)], s_hbm.at[pl.ds(o0, _RPS)])

    return k(y, src1d, dst1d)


_HR = _NP // _H
_HRT = _HR // 16


def _deg_hist(dst1d):
    @functools.partial(
        pl.kernel,
        out_type=jax.ShapeDtypeStruct((2 * _NP * 16,), _f32),
        mesh=_mesh(),
        compiler_params=dataclasses.replace(
            pltpu.CompilerParams(), needs_layout_passes=False),
        scratch_types=[
            pltpu.VMEM((_EB,), jnp.int32),
            pltpu.VMEM((_HR, _H), _f32),
            pltpu.VMEM((_HR,), jnp.int32),
            pltpu.VMEM((_HRT, _H), _f32),
            pltpu.VMEM((_RPS * 16,), _f32),
            pltpu.VMEM_SHARED((_HR, _H), _f32),
        ],
    )
    def k(dst_hbm, d_hbm, dstv, hist, idv, dbuf, bbuf, sdeg):
        c = lax.axis_index("c")
        s = lax.axis_index("s")
        w = s * 2 + c

        @pl.loop(0, _HR)
        def _(i):
            @pl.loop(0, _H // 16)
            def _(j):
                hist[i, pl.ds(j * 16, 16)] = jnp.zeros((16,), _f32)

        @pl.loop(0, _HR // 16)
        def _(i):
            idv[pl.ds(i * 16, 16)] = lax.iota(jnp.int32, 16) + i * 16

        @pl.when(s == 0)
        def _():
            pltpu.sync_copy(hist, sdeg)

        plsc.subcore_barrier()

        ones16 = jnp.ones((16,), _f32)

        @pl.loop(0, (_NBAT + _NW - 1) // _NW)
        def _(t):
            j = w + t * _NW

            @pl.when(j < _NBAT)
            def _():
                e0 = pl.multiple_of(j * _EB, _EB)
                pltpu.sync_copy(dst_hbm.at[pl.ds(e0, _EB)], dstv)

                @pl.loop(0, _EB // 16)
                def _(kk):
                    d16 = dstv[pl.ds(kk * 16, 16)]
                    r = lax.shift_right_logical(d16, 7)
                    cl = lax.bitwise_and(d16, 127)
                    plsc.addupdate_scatter(hist, [r, cl], ones16)

        pltpu.sync_copy(hist, sdeg.at[idv], add=True)
        plsc.subcore_barrier()

        pltpu.sync_copy(sdeg.at[pl.ds(s * _HRT, _HRT)], dbuf)

        @pl.loop(0, _HRT)
        def _(r):
            @pl.loop(0, _H // 16)
            def _(lg):
                v = dbuf[r, pl.ds(lg * 16, 16)]
                base = (r * _H + lg * 16) * 16
                for ll in range(16):
                    bbuf[pl.ds(base + ll * 16, 16)] = jnp.full((16,), v[ll], _f32)

        o0 = pl.multiple_of((c * _NP + s * _RPS) * 16, 8)
        pltpu.sync_copy(bbuf, d_hbm.at[pl.ds(o0, _RPS * 16)])

    return k(dst1d)


def _edge_gather(u, v, e0, e1):
    _TB = _NBL // _NW

    @functools.partial(
        pl.kernel,
        out_type=(jax.ShapeDtypeStruct((_ELP, _H), _f32),
                  jax.ShapeDtypeStruct((_ELP, _H), _f32)),
        mesh=_mesh(),
        scratch_types=[
            pltpu.VMEM((2, _EB), jnp.int32),
            pltpu.VMEM((2, _EB), jnp.int32),
            pltpu.VMEM((2, _EB, _H), _f32),
            pltpu.VMEM((2, _EB, _H), _f32),
            pltpu.SemaphoreType.DMA,
            pltpu.SemaphoreType.DMA,
            pltpu.SemaphoreType.DMA,
            pltpu.SemaphoreType.DMA,
        ],
    )
    def k(u_hbm, v_hbm, e0_hbm, e1_hbm, ug_hbm, vg_hbm,
          i0, i1, ru, rv, sem_i, sem_g0, sem_g1, sem_w):
        sem_g = (sem_g0, sem_g1)
        c = lax.axis_index("c")
        s = lax.axis_index("s")
        w = s * 2 + c

        def boff(t):
            return pl.multiple_of((w + t * _NW) * _EB, _EB)

        def idx_copies(t, b):
            o = boff(t)
            return (pltpu.make_async_copy(e0_hbm.at[pl.ds(o, _EB)],
                                          i0.at[b], sem_i),
                    pltpu.make_async_copy(e1_hbm.at[pl.ds(o, _EB)],
                                          i1.at[b], sem_i))

        def gat_copies(b):
            return (pltpu.make_async_copy(u_hbm.at[i0.at[b]], ru.at[b],
                                          sem_g[b]),
                    pltpu.make_async_copy(v_hbm.at[i1.at[b]], rv.at[b],
                                          sem_g[b]))

        def wr_copies(t, b):
            o = boff(t)
            return (pltpu.make_async_copy(ru.at[b], ug_hbm.at[pl.ds(o, _EB)],
                                          sem_w),
                    pltpu.make_async_copy(rv.at[b], vg_hbm.at[pl.ds(o, _EB)],
                                          sem_w))

        for cp in idx_copies(0, 0):
            cp.start()
        for cp in idx_copies(1, 1):
            cp.start()

        @pl.loop(0, _TB + 1, step=2)
        def _(T):
            for b in (0, 1):
                t = T + b

                @pl.when(t < _TB)
                def _():
                    for cp in idx_copies(t, b):
                        cp.wait()

                @pl.when(t >= 2)
                def _():
                    for cp in wr_copies(t - 2, b):
                        cp.wait()

                @pl.when(t < _TB)
                def _():
                    for cp in gat_copies(b):
                        cp.start()

                @pl.when(t >= 1)
                def _():
                    for cp in gat_copies(1 - b):
                        cp.wait()
                    for cp in wr_copies(t - 1, 1 - b):
                        cp.start()

                    @pl.when(t + 1 < _TB)
                    def _():
                        for cp in idx_copies(t + 1, 1 - b):
                            cp.start()

        for cp in wr_copies(_TB - 1, (_TB - 1) % 2):
            cp.wait()

    return k(u, v, e0, e1)


def _tc_pre(x, W1l, W1r, b1):
    R = 1024

    def body(x_ref, wl_ref, wr_ref, b_ref, y_ref, xr_ref):
        xx = x_ref[...]
        y_ref[...] = jnp.dot(xx, wl_ref[...], preferred_element_type=_f32)
        xr_ref[...] = (jnp.dot(xx, wr_ref[...], preferred_element_type=_f32)
                       + b_ref[...])

    return pl.pallas_call(
        body,
        grid=(_NP // R,),
        in_specs=[pl.BlockSpec((R, _H), lambda i: (i, 0)),
                  pl.BlockSpec((_H, _H), lambda i: (0, 0)),
                  pl.BlockSpec((_H, _H), lambda i: (0, 0)),
                  pl.BlockSpec((1, _H), lambda i: (0, 0))],
        out_specs=[pl.BlockSpec((R, _H), lambda i: (i, 0)),
                   pl.BlockSpec((R, _H), lambda i: (i, 0))],
        out_shape=[jax.ShapeDtypeStruct((_NP, _H), _f32)] * 2,
    )(x, W1l, W1r, b1.reshape(1, _H))


def _tc_mid(s1p, degp, xr1b, W2l, W2r, b2):
    R = 1024

    def body(sp, dp, xr, wl, wr, b, y2, hr):
        ssum = sp[0] + sp[1]
        dg = dp[0, :, 0:1] + dp[1, :, 0:1]
        inv = 1.0 / jnp.maximum(dg, 1.0)
        h = jnp.maximum(ssum * inv + xr[...], 0.0)
        y2[...] = jnp.dot(h, wl[...], preferred_element_type=_f32)
        hr[...] = jnp.dot(h, wr[...], preferred_element_type=_f32) + b[...]

    return pl.pallas_call(
        body,
        grid=(_NP // R,),
        in_specs=[pl.BlockSpec((2, R, _H), lambda i: (0, i, 0)),
                  pl.BlockSpec((2, R, 16), lambda i: (0, i, 0)),
                  pl.BlockSpec((R, _H), lambda i: (i, 0)),
                  pl.BlockSpec((_H, _H), lambda i: (0, 0)),
                  pl.BlockSpec((_H, _H), lambda i: (0, 0)),
                  pl.BlockSpec((1, _H), lambda i: (0, 0))],
        out_specs=[pl.BlockSpec((R, _H), lambda i: (i, 0)),
                   pl.BlockSpec((R, _H), lambda i: (i, 0))],
        out_shape=[jax.ShapeDtypeStruct((_NP, _H), _f32)] * 2,
    )(s1p, degp, xr1b, W2l, W2r, b2.reshape(1, _H))


def _tc_z(s2p, degp, hr2b, Wd1, bd1):
    R = 1024

    def body(sp, dp, hr, wd, b, u, v):
        ssum = sp[0] + sp[1]
        dg = dp[0, :, 0:1] + dp[1, :, 0:1]
        inv = 1.0 / jnp.maximum(dg, 1.0)
        z = ssum * inv + hr[...]
        wa = wd[0:_H, :]
        wb = wd[_H:2 * _H, :]
        u[...] = jnp.dot(z, wa, preferred_element_type=_f32) + b[...]
        v[...] = jnp.dot(z, wb, preferred_element_type=_f32)

    return pl.pallas_call(
        body,
        grid=(_NP // R,),
        in_specs=[pl.BlockSpec((2, R, _H), lambda i: (0, i, 0)),
                  pl.BlockSpec((2, R, 16), lambda i: (0, i, 0)),
                  pl.BlockSpec((R, _H), lambda i: (i, 0)),
                  pl.BlockSpec((2 * _H, _H), lambda i: (0, 0)),
                  pl.BlockSpec((1, _H), lambda i: (0, 0))],
        out_specs=[pl.BlockSpec((R, _H), lambda i: (i, 0)),
                   pl.BlockSpec((R, _H), lambda i: (i, 0))],
        out_shape=[jax.ShapeDtypeStruct((_NP, _H), _f32)] * 2,
    )(s2p, degp, hr2b, Wd1, bd1.reshape(1, _H))


def _tc_dec(ug, vg, wd2r, bd2):
    R = 1024

    def body(a, b, w, bb, o):
        t = jnp.maximum(a[...] + b[...], 0.0)
        val = jnp.sum(t * w[...], axis=1) + bb[0, 0]
        o[...] = val.reshape(R // _H, _H)

    return pl.pallas_call(
        body,
        grid=(_ELP // R,),
        in_specs=[pl.BlockSpec((R, _H), lambda i: (i, 0)),
                  pl.BlockSpec((R, _H), lambda i: (i, 0)),
                  pl.BlockSpec((1, _H), lambda i: (0, 0)),
                  pl.BlockSpec((1, 1), lambda i: (0, 0))],
        out_specs=pl.BlockSpec((R // _H, _H), lambda i: (i, 0)),
        out_shape=jax.ShapeDtypeStruct((_ELP // _H, _H), _f32),
    )(ug, vg, wd2r, bd2)


def kernel(x, edge_index, edge_label_index,
           W1l, b1, W1r, W2l, b2, W2r, Wd1, bd1, Wd2, bd2):
    xp = jnp.pad(x, ((0, _NP - _N), (0, 0)))
    src1d = edge_index[0]
    dst1d = edge_index[1]

    y1, xr1b = _tc_pre(xp, W1l, W1r, b1)
    degf = _deg_hist(dst1d)
    s1f = _segsum(y1, src1d, dst1d)
    s1p = s1f.reshape(2, _NP, _H)
    degp = degf.reshape(2, _NP, 16)

    y2, hr2b = _tc_mid(s1p, degp, xr1b, W2l, W2r, b2)
    s2f = _segsum(y2, src1d, dst1d)
    s2p = s2f.reshape(2, _NP, _H)

    u, v = _tc_z(s2p, degp, hr2b, Wd1, bd1)

    pad = _ELP - _EL
    e0 = jnp.pad(edge_label_index[0], (0, pad))
    e1 = jnp.pad(edge_label_index[1], (0, pad))
    ug, vg = _edge_gather(u, v, e0, e1)

    out2d = _tc_dec(ug, vg, Wd2.reshape(1, _H), bd2.reshape(1, 1))
    return out2d.reshape(-1)[:_EL]


# speedup vs baseline: 6.8916x; 1.4388x over previous
import dataclasses
import functools

import jax
import jax.numpy as jnp
from jax import lax
from jax.experimental import pallas as pl
from jax.experimental.pallas import tpu as pltpu
from jax.experimental.pallas import tpu_sc as plsc

_N = 10000
_NP = 10240
_H = 128
_E = 320000
_EL = 100000
_EB = 128
_NBAT = _E // _EB
_NW = 32
_RPS = _NP // 16
_ELP = 102400
_NBL = _ELP // _EB

_f32 = jnp.float32


def _mesh():
    return plsc.VectorSubcoreMesh(core_axis_name="c", subcore_axis_name="s")


def _segsum(y, src1d, dst1d):
    _TB = (_NBAT + _NW - 1) // _NW

    @functools.partial(
        pl.kernel,
        out_type=jax.ShapeDtypeStruct((2 * _NP, _H), _f32),
        mesh=_mesh(),
        scratch_types=[
            pltpu.VMEM((2, _EB), jnp.int32),
            pltpu.VMEM((2, _EB), jnp.int32),
            pltpu.VMEM((2, _EB, _H), _f32),
            pltpu.VMEM_SHARED((_NP, _H), _f32),
            pltpu.SemaphoreType.DMA,
            pltpu.SemaphoreType.DMA,
            pltpu.SemaphoreType.DMA,
            pltpu.SemaphoreType.DMA,
        ],
    )
    def k(y_hbm, src_hbm, dst_hbm, s_hbm,
          srcv, dstv, rows, acc, sem_i, sem_g0, sem_g1, sem_s):
        sem_g = (sem_g0, sem_g1)
        c = lax.axis_index("c")
        s = lax.axis_index("s")
        w = s * 2 + c
        r0 = pl.multiple_of(s * _RPS, _RPS)

        def vld(t):
            return (w + t * _NW) < _NBAT

        def idx_copies(t, b):
            o = pl.multiple_of((w + t * _NW) * _EB, _EB)
            return (pltpu.make_async_copy(src_hbm.at[pl.ds(o, _EB)],
                                          srcv.at[b], sem_i),
                    pltpu.make_async_copy(dst_hbm.at[pl.ds(o, _EB)],
                                          dstv.at[b], sem_i))

        def gat_copy(b):
            return pltpu.make_async_copy(y_hbm.at[srcv.at[b]], rows.at[b],
                                         sem_g[b])

        def sct_copy(b):
            return pltpu.make_async_copy(rows.at[b], acc.at[dstv.at[b]],
                                         sem_s)

        @pl.loop(0, _EB)
        def _(i):
            @pl.loop(0, _H // 16)
            def _(j):
                rows[0, i, pl.ds(j * 16, 16)] = jnp.zeros((16,), _f32)

        @pl.loop(0, _RPS // _EB)
        def _(i):
            pltpu.sync_copy(rows.at[0], acc.at[pl.ds(r0 + i * _EB, _EB)])

        plsc.subcore_barrier()

        for cp in idx_copies(0, 0):
            cp.start()
        for cp in idx_copies(1, 1):
            cp.start()

        @pl.loop(0, _TB + 2, step=2)
        def _(T):
            for b in (0, 1):
                t = T + b

                @pl.when((t < _TB) & vld(t))
                def _():
                    for cp in idx_copies(t, b):
                        cp.wait()

                @pl.when((t >= 2) & vld(t - 2))
                def _():
                    sct_copy(b).wait()

                @pl.when((t < _TB) & vld(t))
                def _():
                    gat_copy(b).start()

                @pl.when((t >= 1) & (t - 1 < _TB) & vld(t - 1))
                def _():
                    gat_copy(1 - b).wait()
                    sct_copy(1 - b).start(add=True)

                    @pl.when((t + 1 < _TB) & vld(t + 1))
                    def _():
                        for cp in idx_copies(t + 1, 1 - b):
                            cp.start()

        plsc.subcore_barrier()
        o0 = pl.multiple_of(c * _NP + r0, _RPS)
        pltpu.sync_copy(acc.at[pl.ds(r0, _RPS)], s_hbm.at[pl.ds(o0, _RPS)])

    return k(y, src1d, dst1d)


_HR = _NP // _H
_HRT = _HR // 16


def _deg_hist(dst1d):
    @functools.partial(
        pl.kernel,
        out_type=jax.ShapeDtypeStruct((2 * _NP * 16,), _f32),
        mesh=_mesh(),
        compiler_params=dataclasses.replace(
            pltpu.CompilerParams(), needs_layout_passes=False),
        scratch_types=[
            pltpu.VMEM((_EB,), jnp.int32),
            pltpu.VMEM((_HR, _H), _f32),
            pltpu.VMEM((_HR,), jnp.int32),
            pltpu.VMEM((_HRT, _H), _f32),
            pltpu.VMEM((_RPS * 16,), _f32),
            pltpu.VMEM_SHARED((_HR, _H), _f32),
        ],
    )
    def k(dst_hbm, d_hbm, dstv, hist, idv, dbuf, bbuf, sdeg):
        c = lax.axis_index("c")
        s = lax.axis_index("s")
        w = s * 2 + c

        @pl.loop(0, _HR)
        def _(i):
            @pl.loop(0, _H // 16)
            def _(j):
                hist[i, pl.ds(j * 16, 16)] = jnp.zeros((16,), _f32)

        @pl.loop(0, _HR // 16)
        def _(i):
            idv[pl.ds(i * 16, 16)] = lax.iota(jnp.int32, 16) + i * 16

        @pl.when(s == 0)
        def _():
            pltpu.sync_copy(hist, sdeg)

        plsc.subcore_barrier()

        ones16 = jnp.ones((16,), _f32)

        @pl.loop(0, (_NBAT + _NW - 1) // _NW)
        def _(t):
            j = w + t * _NW

            @pl.when(j < _NBAT)
            def _():
                e0 = pl.multiple_of(j * _EB, _EB)
                pltpu.sync_copy(dst_hbm.at[pl.ds(e0, _EB)], dstv)

                @pl.loop(0, _EB // 16)
                def _(kk):
                    d16 = dstv[pl.ds(kk * 16, 16)]
                    r = lax.shift_right_logical(d16, 7)
                    cl = lax.bitwise_and(d16, 127)
                    plsc.addupdate_scatter(hist, [r, cl], ones16)

        pltpu.sync_copy(hist, sdeg.at[idv], add=True)
        plsc.subcore_barrier()

        pltpu.sync_copy(sdeg.at[pl.ds(s * _HRT, _HRT)], dbuf)

        @pl.loop(0, _HRT)
        def _(r):
            @pl.loop(0, _H // 16)
            def _(lg):
                v = dbuf[r, pl.ds(lg * 16, 16)]
                base = (r * _H + lg * 16) * 16
                for ll in range(16):
                    bbuf[pl.ds(base + ll * 16, 16)] = jnp.full((16,), v[ll], _f32)

        o0 = pl.multiple_of((c * _NP + s * _RPS) * 16, 8)
        pltpu.sync_copy(bbuf, d_hbm.at[pl.ds(o0, _RPS * 16)])

    return k(dst1d)


def _edge_gather(u, v, e0, e1):
    _TB = _NBL // _NW

    @functools.partial(
        pl.kernel,
        out_type=(jax.ShapeDtypeStruct((_ELP, _H), _f32),
                  jax.ShapeDtypeStruct((_ELP, _H), _f32)),
        mesh=_mesh(),
        scratch_types=[
            pltpu.VMEM((2, _EB), jnp.int32),
            pltpu.VMEM((2, _EB), jnp.int32),
            pltpu.VMEM((2, _EB, _H), _f32),
            pltpu.VMEM((2, _EB, _H), _f32),
            pltpu.SemaphoreType.DMA,
            pltpu.SemaphoreType.DMA,
            pltpu.SemaphoreType.DMA,
            pltpu.SemaphoreType.DMA,
        ],
    )
    def k(u_hbm, v_hbm, e0_hbm, e1_hbm, ug_hbm, vg_hbm,
          i0, i1, ru, rv, sem_i, sem_g0, sem_g1, sem_w):
        sem_g = (sem_g0, sem_g1)
        c = lax.axis_index("c")
        s = lax.axis_index("s")
        w = s * 2 + c

        def boff(t):
            return pl.multiple_of((w + t * _NW) * _EB, _EB)

        def idx_copies(t, b):
            o = boff(t)
            return (pltpu.make_async_copy(e0_hbm.at[pl.ds(o, _EB)],
                                          i0.at[b], sem_i),
                    pltpu.make_async_copy(e1_hbm.at[pl.ds(o, _EB)],
                                          i1.at[b], sem_i))

        def gat_copies(b):
            return (pltpu.make_async_copy(u_hbm.at[i0.at[b]], ru.at[b],
                                          sem_g[b]),
                    pltpu.make_async_copy(v_hbm.at[i1.at[b]], rv.at[b],
                                          sem_g[b]))

        def wr_copies(t, b):
            o = boff(t)
            return (pltpu.make_async_copy(ru.at[b], ug_hbm.at[pl.ds(o, _EB)],
                                          sem_w),
                    pltpu.make_async_copy(rv.at[b], vg_hbm.at[pl.ds(o, _EB)],
                                          sem_w))

        for cp in idx_copies(0, 0):
            cp.start()
        for cp in idx_copies(1, 1):
            cp.start()

        @pl.loop(0, _TB + 1, step=2)
        def _(T):
            for b in (0, 1):
                t = T + b

                @pl.when(t < _TB)
                def _():
                    for cp in idx_copies(t, b):
                        cp.wait()

                @pl.when(t >= 2)
                def _():
                    for cp in wr_copies(t - 2, b):
                        cp.wait()

                @pl.when(t < _TB)
                def _():
                    for cp in gat_copies(b):
                        cp.start()

                @pl.when(t >= 1)
                def _():
                    for cp in gat_copies(1 - b):
                        cp.wait()
                    for cp in wr_copies(t - 1, 1 - b):
                        cp.start()

                    @pl.when(t + 1 < _TB)
                    def _():
                        for cp in idx_copies(t + 1, 1 - b):
                            cp.start()

        for cp in wr_copies(_TB - 1, (_TB - 1) % 2):
            cp.wait()

    return k(u, v, e0, e1)


def _tc_pre(x, W1l, W1r, b1):
    R = 1024

    def body(x_ref, wl_ref, wr_ref, b_ref, y_ref, xr_ref):
        xx = x_ref[...]
        y_ref[...] = jnp.dot(xx, wl_ref[...], preferred_element_type=_f32)
        xr_ref[...] = (jnp.dot(xx, wr_ref[...], preferred_element_type=_f32)
                       + b_ref[...])

    return pl.pallas_call(
        body,
        grid=(_NP // R,),
        in_specs=[pl.BlockSpec((R, _H), lambda i: (i, 0)),
                  pl.BlockSpec((_H, _H), lambda i: (0, 0)),
                  pl.BlockSpec((_H, _H), lambda i: (0, 0)),
                  pl.BlockSpec((1, _H), lambda i: (0, 0))],
        out_specs=[pl.BlockSpec((R, _H), lambda i: (i, 0)),
                   pl.BlockSpec((R, _H), lambda i: (i, 0))],
        out_shape=[jax.ShapeDtypeStruct((_NP, _H), _f32)] * 2,
    )(x, W1l, W1r, b1.reshape(1, _H))


def _tc_mid(s1p, degp, xr1b, W2l, W2r, b2):
    R = 1024

    def body(sp, dp, xr, wl, wr, b, y2, hr):
        ssum = sp[0] + sp[1]
        dg = dp[0, :, 0:1] + dp[1, :, 0:1]
        inv = 1.0 / jnp.maximum(dg, 1.0)
        h = jnp.maximum(ssum * inv + xr[...], 0.0)
        y2[...] = jnp.dot(h, wl[...], preferred_element_type=_f32)
        hr[...] = jnp.dot(h, wr[...], preferred_element_type=_f32) + b[...]

    return pl.pallas_call(
        body,
        grid=(_NP // R,),
        in_specs=[pl.BlockSpec((2, R, _H), lambda i: (0, i, 0)),
                  pl.BlockSpec((2, R, 16), lambda i: (0, i, 0)),
                  pl.BlockSpec((R, _H), lambda i: (i, 0)),
                  pl.BlockSpec((_H, _H), lambda i: (0, 0)),
                  pl.BlockSpec((_H, _H), lambda i: (0, 0)),
                  pl.BlockSpec((1, _H), lambda i: (0, 0))],
        out_specs=[pl.BlockSpec((R, _H), lambda i: (i, 0)),
                   pl.BlockSpec((R, _H), lambda i: (i, 0))],
        out_shape=[jax.ShapeDtypeStruct((_NP, _H), _f32)] * 2,
    )(s1p, degp, xr1b, W2l, W2r, b2.reshape(1, _H))


def _tc_z(s2p, degp, hr2b, Wd1, bd1):
    R = 1024

    def body(sp, dp, hr, wd, b, u, v):
        ssum = sp[0] + sp[1]
        dg = dp[0, :, 0:1] + dp[1, :, 0:1]
        inv = 1.0 / jnp.maximum(dg, 1.0)
        z = ssum * inv + hr[...]
        wa = wd[0:_H, :]
        wb = wd[_H:2 * _H, :]
        u[...] = jnp.dot(z, wa, preferred_element_type=_f32) + b[...]
        v[...] = jnp.dot(z, wb, preferred_element_type=_f32)

    return pl.pallas_call(
        body,
        grid=(_NP // R,),
        in_specs=[pl.BlockSpec((2, R, _H), lambda i: (0, i, 0)),
                  pl.BlockSpec((2, R, 16), lambda i: (0, i, 0)),
                  pl.BlockSpec((R, _H), lambda i: (i, 0)),
                  pl.BlockSpec((2 * _H, _H), lambda i: (0, 0)),
                  pl.BlockSpec((1, _H), lambda i: (0, 0))],
        out_specs=[pl.BlockSpec((R, _H), lambda i: (i, 0)),
                   pl.BlockSpec((R, _H), lambda i: (i, 0))],
        out_shape=[jax.ShapeDtypeStruct((_NP, _H), _f32)] * 2,
    )(s2p, degp, hr2b, Wd1, bd1.reshape(1, _H))


def _tc_dec(ug, vg, wd2r, bd2):
    R = 1024

    def body(a, b, w, bb, o):
        t = jnp.maximum(a[...] + b[...], 0.0)
        val = jnp.sum(t * w[...], axis=1) + bb[0, 0]
        o[...] = val.reshape(R // _H, _H)

    return pl.pallas_call(
        body,
        grid=(_ELP // R,),
        in_specs=[pl.BlockSpec((R, _H), lambda i: (i, 0)),
                  pl.BlockSpec((R, _H), lambda i: (i, 0)),
                  pl.BlockSpec((1, _H), lambda i: (0, 0)),
                  pl.BlockSpec((1, 1), lambda i: (0, 0))],
        out_specs=pl.BlockSpec((R // _H, _H), lambda i: (i, 0)),
        out_shape=jax.ShapeDtypeStruct((_ELP // _H, _H), _f32),
    )(ug, vg, wd2r, bd2)


def kernel(x, edge_index, edge_label_index,
           W1l, b1, W1r, W2l, b2, W2r, Wd1, bd1, Wd2, bd2):
    xp = jnp.pad(x, ((0, _NP - _N), (0, 0)))
    src1d = edge_index[0]
    dst1d = edge_index[1]

    y1, xr1b = _tc_pre(xp, W1l, W1r, b1)
    degf = _deg_hist(dst1d)
    s1f = _segsum(y1, src1d, dst1d)
    s1p = s1f.reshape(2, _NP, _H)
    degp = degf.reshape(2, _NP, 16)

    y2, hr2b = _tc_mid(s1p, degp, xr1b, W2l, W2r, b2)
    s2f = _segsum(y2, src1d, dst1d)
    s2p = s2f.reshape(2, _NP, _H)

    u, v = _tc_z(s2p, degp, hr2b, Wd1, bd1)

    pad = _ELP - _EL
    e0 = jnp.pad(edge_label_index[0], (0, pad))
    e1 = jnp.pad(edge_label_index[1], (0, pad))
    ug, vg = _edge_gather(u, v, e0, e1)

    out2d = _tc_dec(ug, vg, Wd2.reshape(1, _H), bd2.reshape(1, 1))
    return out2d.reshape(-1)[:_EL]


# speedup vs baseline: 8.6416x; 1.2539x over previous
import dataclasses
import functools

import jax
import jax.numpy as jnp
from jax import lax
from jax.experimental import pallas as pl
from jax.experimental.pallas import tpu as pltpu
from jax.experimental.pallas import tpu_sc as plsc

_N = 10000
_NP = 10240
_H = 128
_E = 320000
_EL = 100000
_EB = 128
_NBAT = _E // _EB
_NW = 32
_RPS = _NP // 16
_ELP = 102400
_NBL = _ELP // _EB

_f32 = jnp.float32


def _mesh():
    return plsc.VectorSubcoreMesh(core_axis_name="c", subcore_axis_name="s")


def _segsum(y, src1d, dst1d):
    _TB = (_NBAT + _NW - 1) // _NW

    @functools.partial(
        pl.kernel,
        out_type=jax.ShapeDtypeStruct((2 * _NP, _H), _f32),
        mesh=_mesh(),
        scratch_types=[
            pltpu.VMEM((2, _EB), jnp.int32),
            pltpu.VMEM((2, _EB), jnp.int32),
            pltpu.VMEM((2, _EB, _H), _f32),
            pltpu.VMEM_SHARED((_NP, _H), _f32),
            pltpu.SemaphoreType.DMA,
            pltpu.SemaphoreType.DMA,
            pltpu.SemaphoreType.DMA,
            pltpu.SemaphoreType.DMA,
        ],
    )
    def k(y_hbm, src_hbm, dst_hbm, s_hbm,
          srcv, dstv, rows, acc, sem_i, sem_g0, sem_g1, sem_s):
        sem_g = (sem_g0, sem_g1)
        c = lax.axis_index("c")
        s = lax.axis_index("s")
        w = s * 2 + c
        r0 = pl.multiple_of(s * _RPS, _RPS)

        def vld(t):
            return (w + t * _NW) < _NBAT

        def idx_copies(t, b):
            o = pl.multiple_of((w + t * _NW) * _EB, _EB)
            return (pltpu.make_async_copy(src_hbm.at[pl.ds(o, _EB)],
                                          srcv.at[b], sem_i),
                    pltpu.make_async_copy(dst_hbm.at[pl.ds(o, _EB)],
                                          dstv.at[b], sem_i))

        def gat_copy(b):
            return pltpu.make_async_copy(y_hbm.at[srcv.at[b]], rows.at[b],
                                         sem_g[b])

        def sct_copy(b):
            return pltpu.make_async_copy(rows.at[b], acc.at[dstv.at[b]],
                                         sem_s)

        @pl.loop(0, _EB)
        def _(i):
            @pl.loop(0, _H // 16)
            def _(j):
                rows[0, i, pl.ds(j * 16, 16)] = jnp.zeros((16,), _f32)

        @pl.loop(0, _RPS // _EB)
        def _(i):
            pltpu.sync_copy(rows.at[0], acc.at[pl.ds(r0 + i * _EB, _EB)])

        plsc.subcore_barrier()

        for cp in idx_copies(0, 0):
            cp.start()
        for cp in idx_copies(1, 1):
            cp.start()

        @pl.loop(0, _TB + 2, step=2)
        def _(T):
            for b in (0, 1):
                t = T + b

                @pl.when((t < _TB) & vld(t))
                def _():
                    for cp in idx_copies(t, b):
                        cp.wait()

                @pl.when((t >= 2) & vld(t - 2))
                def _():
                    sct_copy(b).wait()

                @pl.when((t < _TB) & vld(t))
                def _():
                    gat_copy(b).start()

                @pl.when((t >= 1) & (t - 1 < _TB) & vld(t - 1))
                def _():
                    gat_copy(1 - b).wait()
                    sct_copy(1 - b).start(add=True)

                    @pl.when((t + 1 < _TB) & vld(t + 1))
                    def _():
                        for cp in idx_copies(t + 1, 1 - b):
                            cp.start()

        plsc.subcore_barrier()
        o0 = pl.multiple_of(c * _NP + r0, _RPS)
        pltpu.sync_copy(acc.at[pl.ds(r0, _RPS)], s_hbm.at[pl.ds(o0, _RPS)])

    return k(y, src1d, dst1d)


_HR = _NP // _H
_HRT = _HR // 16


def _deg_hist(dst1d):
    @functools.partial(
        pl.kernel,
        out_type=jax.ShapeDtypeStruct((2 * _NP * 16,), _f32),
        mesh=_mesh(),
        compiler_params=dataclasses.replace(
            pltpu.CompilerParams(), needs_layout_passes=False),
        scratch_types=[
            pltpu.VMEM((_EB,), jnp.int32),
            pltpu.VMEM((_HR, _H), _f32),
            pltpu.VMEM((_HR,), jnp.int32),
            pltpu.VMEM((_HRT, _H), _f32),
            pltpu.VMEM((_RPS * 16,), _f32),
            pltpu.VMEM_SHARED((_HR, _H), _f32),
        ],
    )
    def k(dst_hbm, d_hbm, dstv, hist, idv, dbuf, bbuf, sdeg):
        c = lax.axis_index("c")
        s = lax.axis_index("s")
        w = s * 2 + c

        @pl.loop(0, _HR)
        def _(i):
            @pl.loop(0, _H // 16)
            def _(j):
                hist[i, pl.ds(j * 16, 16)] = jnp.zeros((16,), _f32)

        @pl.loop(0, _HR // 16)
        def _(i):
            idv[pl.ds(i * 16, 16)] = lax.iota(jnp.int32, 16) + i * 16

        @pl.when(s == 0)
        def _():
            pltpu.sync_copy(hist, sdeg)

        plsc.subcore_barrier()

        ones16 = jnp.ones((16,), _f32)

        @pl.loop(0, (_NBAT + _NW - 1) // _NW)
        def _(t):
            j = w + t * _NW

            @pl.when(j < _NBAT)
            def _():
                e0 = pl.multiple_of(j * _EB, _EB)
                pltpu.sync_copy(dst_hbm.at[pl.ds(e0, _EB)], dstv)

                @pl.loop(0, _EB // 16)
                def _(kk):
                    d16 = dstv[pl.ds(kk * 16, 16)]
                    r = lax.shift_right_logical(d16, 7)
                    cl = lax.bitwise_and(d16, 127)
                    plsc.addupdate_scatter(hist, [r, cl], ones16)

        pltpu.sync_copy(hist, sdeg.at[idv], add=True)
        plsc.subcore_barrier()

        pltpu.sync_copy(sdeg.at[pl.ds(s * _HRT, _HRT)], dbuf)

        @pl.loop(0, _HRT)
        def _(r):
            @pl.loop(0, _H // 16)
            def _(lg):
                v = dbuf[r, pl.ds(lg * 16, 16)]
                base = (r * _H + lg * 16) * 16
                for ll in range(16):
                    bbuf[pl.ds(base + ll * 16, 16)] = jnp.full((16,), v[ll], _f32)

        o0 = pl.multiple_of((c * _NP + s * _RPS) * 16, 8)
        pltpu.sync_copy(bbuf, d_hbm.at[pl.ds(o0, _RPS * 16)])

    return k(dst1d)


def _edge_decode(u, v, e0, e1, wd2f, bd2p):
    _TB = _NBL // _NW

    @functools.partial(
        pl.kernel,
        out_type=jax.ShapeDtypeStruct((_ELP,), _f32),
        mesh=_mesh(),
        compiler_params=dataclasses.replace(
            pltpu.CompilerParams(), needs_layout_passes=False),
        scratch_types=[
            pltpu.VMEM((2, _EB), jnp.int32),
            pltpu.VMEM((2, _EB), jnp.int32),
            pltpu.VMEM((2, _EB, _H), _f32),
            pltpu.VMEM((2, _EB, _H), _f32),
            pltpu.VMEM((2, _EB), _f32),
            pltpu.VMEM((_H,), _f32),
            pltpu.VMEM((16,), _f32),
            pltpu.VMEM((16, 16), _f32),
            pltpu.SemaphoreType.DMA,
            pltpu.SemaphoreType.DMA,
            pltpu.SemaphoreType.DMA,
            pltpu.SemaphoreType.DMA,
        ],
    )
    def k(u_hbm, v_hbm, e0_hbm, e1_hbm, w_hbm, b_hbm, o_hbm,
          i0, i1, ru, rv, obuf, wv, bv, macc,
          sem_i, sem_g0, sem_g1, sem_w):
        sem_g = (sem_g0, sem_g1)
        c = lax.axis_index("c")
        s = lax.axis_index("s")
        w = s * 2 + c

        pltpu.sync_copy(w_hbm, wv)
        pltpu.sync_copy(b_hbm, bv)
        wch = [wv[pl.ds(kk * 16, 16)] for kk in range(_H // 16)]
        bd2s = bv[...][0]
        iota16 = lax.iota(jnp.int32, 16)

        def boff(t):
            return pl.multiple_of((w + t * _NW) * _EB, _EB)

        def idx_copies(t, b):
            o = boff(t)
            return (pltpu.make_async_copy(e0_hbm.at[pl.ds(o, _EB)],
                                          i0.at[b], sem_i),
                    pltpu.make_async_copy(e1_hbm.at[pl.ds(o, _EB)],
                                          i1.at[b], sem_i))

        def gat_copies(b):
            return (pltpu.make_async_copy(u_hbm.at[i0.at[b]], ru.at[b],
                                          sem_g[b]),
                    pltpu.make_async_copy(v_hbm.at[i1.at[b]], rv.at[b],
                                          sem_g[b]))

        def wr_copy(t, b):
            return pltpu.make_async_copy(obuf.at[b],
                                         o_hbm.at[pl.ds(boff(t), _EB)], sem_w)

        def compute(b):
            @pl.loop(0, _EB // 16)
            def _(g):
                @pl.loop(0, 16)
                def _(i):
                    r = g * 16 + i
                    acc = jnp.zeros((16,), _f32)
                    for kk in range(_H // 16):
                        uu = ru[b, r, pl.ds(kk * 16, 16)]
                        vv = rv[b, r, pl.ds(kk * 16, 16)]
                        acc = acc + jnp.maximum(uu + vv, 0.0) * wch[kk]
                    macc[i, :] = acc

                o = jnp.full((16,), bd2s, _f32)
                for l in range(16):
                    o = o + plsc.load_gather(
                        macc, [iota16, jnp.full((16,), l, jnp.int32)])
                obuf[b, pl.ds(g * 16, 16)] = o

        for cp in idx_copies(0, 0):
            cp.start()
        for cp in idx_copies(1, 1):
            cp.start()

        @pl.loop(0, _TB + 1, step=2)
        def _(T):
            for b in (0, 1):
                t = T + b

                @pl.when(t < _TB)
                def _():
                    for cp in idx_copies(t, b):
                        cp.wait()

                @pl.when(t >= 2)
                def _():
                    wr_copy(t - 2, b).wait()

                @pl.when(t < _TB)
                def _():
                    for cp in gat_copies(b):
                        cp.start()

                @pl.when(t >= 1)
                def _():
                    for cp in gat_copies(1 - b):
                        cp.wait()
                    compute(1 - b)
                    wr_copy(t - 1, 1 - b).start()

                    @pl.when(t + 1 < _TB)
                    def _():
                        for cp in idx_copies(t + 1, 1 - b):
                            cp.start()

        wr_copy(_TB - 1, (_TB - 1) % 2).wait()

    return k(u, v, e0, e1, wd2f, bd2p)


def _tc_pre(x, W1l, W1r, b1):
    R = 1024

    def body(x_ref, wl_ref, wr_ref, b_ref, y_ref, xr_ref):
        xx = x_ref[...]
        y_ref[...] = jnp.dot(xx, wl_ref[...], preferred_element_type=_f32)
        xr_ref[...] = (jnp.dot(xx, wr_ref[...], preferred_element_type=_f32)
                       + b_ref[...])

    return pl.pallas_call(
        body,
        grid=(_NP // R,),
        in_specs=[pl.BlockSpec((R, _H), lambda i: (i, 0)),
                  pl.BlockSpec((_H, _H), lambda i: (0, 0)),
                  pl.BlockSpec((_H, _H), lambda i: (0, 0)),
                  pl.BlockSpec((1, _H), lambda i: (0, 0))],
        out_specs=[pl.BlockSpec((R, _H), lambda i: (i, 0)),
                   pl.BlockSpec((R, _H), lambda i: (i, 0))],
        out_shape=[jax.ShapeDtypeStruct((_NP, _H), _f32)] * 2,
    )(x, W1l, W1r, b1.reshape(1, _H))


def _tc_mid(s1p, degp, xr1b, W2l, W2r, b2):
    R = 1024

    def body(sp, dp, xr, wl, wr, b, y2, hr):
        ssum = sp[0] + sp[1]
        dg = dp[0, :, 0:1] + dp[1, :, 0:1]
        inv = 1.0 / jnp.maximum(dg, 1.0)
        h = jnp.maximum(ssum * inv + xr[...], 0.0)
        y2[...] = jnp.dot(h, wl[...], preferred_element_type=_f32)
        hr[...] = jnp.dot(h, wr[...], preferred_element_type=_f32) + b[...]

    return pl.pallas_call(
        body,
        grid=(_NP // R,),
        in_specs=[pl.BlockSpec((2, R, _H), lambda i: (0, i, 0)),
                  pl.BlockSpec((2, R, 16), lambda i: (0, i, 0)),
                  pl.BlockSpec((R, _H), lambda i: (i, 0)),
                  pl.BlockSpec((_H, _H), lambda i: (0, 0)),
                  pl.BlockSpec((_H, _H), lambda i: (0, 0)),
                  pl.BlockSpec((1, _H), lambda i: (0, 0))],
        out_specs=[pl.BlockSpec((R, _H), lambda i: (i, 0)),
                   pl.BlockSpec((R, _H), lambda i: (i, 0))],
        out_shape=[jax.ShapeDtypeStruct((_NP, _H), _f32)] * 2,
    )(s1p, degp, xr1b, W2l, W2r, b2.reshape(1, _H))


def _tc_z(s2p, degp, hr2b, Wd1, bd1):
    R = 1024

    def body(sp, dp, hr, wd, b, u, v):
        ssum = sp[0] + sp[1]
        dg = dp[0, :, 0:1] + dp[1, :, 0:1]
        inv = 1.0 / jnp.maximum(dg, 1.0)
        z = ssum * inv + hr[...]
        wa = wd[0:_H, :]
        wb = wd[_H:2 * _H, :]
        u[...] = jnp.dot(z, wa, preferred_element_type=_f32) + b[...]
        v[...] = jnp.dot(z, wb, preferred_element_type=_f32)

    return pl.pallas_call(
        body,
        grid=(_NP // R,),
        in_specs=[pl.BlockSpec((2, R, _H), lambda i: (0, i, 0)),
                  pl.BlockSpec((2, R, 16), lambda i: (0, i, 0)),
                  pl.BlockSpec((R, _H), lambda i: (i, 0)),
                  pl.BlockSpec((2 * _H, _H), lambda i: (0, 0)),
                  pl.BlockSpec((1, _H), lambda i: (0, 0))],
        out_specs=[pl.BlockSpec((R, _H), lambda i: (i, 0)),
                   pl.BlockSpec((R, _H), lambda i: (i, 0))],
        out_shape=[jax.ShapeDtypeStruct((_NP, _H), _f32)] * 2,
    )(s2p, degp, hr2b, Wd1, bd1.reshape(1, _H))


def _tc_dec(ug, vg, wd2r, bd2):
    R = 1024

    def body(a, b, w, bb, o):
        t = jnp.maximum(a[...] + b[...], 0.0)
        val = jnp.sum(t * w[...], axis=1) + bb[0, 0]
        o[...] = val.reshape(R // _H, _H)

    return pl.pallas_call(
        body,
        grid=(_ELP // R,),
        in_specs=[pl.BlockSpec((R, _H), lambda i: (i, 0)),
                  pl.BlockSpec((R, _H), lambda i: (i, 0)),
                  pl.BlockSpec((1, _H), lambda i: (0, 0)),
                  pl.BlockSpec((1, 1), lambda i: (0, 0))],
        out_specs=pl.BlockSpec((R // _H, _H), lambda i: (i, 0)),
        out_shape=jax.ShapeDtypeStruct((_ELP // _H, _H), _f32),
    )(ug, vg, wd2r, bd2)


def kernel(x, edge_index, edge_label_index,
           W1l, b1, W1r, W2l, b2, W2r, Wd1, bd1, Wd2, bd2):
    xp = jnp.pad(x, ((0, _NP - _N), (0, 0)))
    src1d = edge_index[0]
    dst1d = edge_index[1]

    y1, xr1b = _tc_pre(xp, W1l, W1r, b1)
    degf = _deg_hist(dst1d)
    s1f = _segsum(y1, src1d, dst1d)
    s1p = s1f.reshape(2, _NP, _H)
    degp = degf.reshape(2, _NP, 16)

    y2, hr2b = _tc_mid(s1p, degp, xr1b, W2l, W2r, b2)
    s2f = _segsum(y2, src1d, dst1d)
    s2p = s2f.reshape(2, _NP, _H)

    u, v = _tc_z(s2p, degp, hr2b, Wd1, bd1)

    pad = _ELP - _EL
    e0 = jnp.pad(edge_label_index[0], (0, pad))
    e1 = jnp.pad(edge_label_index[1], (0, pad))
    out = _edge_decode(u, v, e0, e1, Wd2.reshape(_H),
                       jnp.pad(bd2, (0, 15)))
    return out[:_EL]


# speedup vs baseline: 9.0242x; 1.0443x over previous
import dataclasses
import functools

import jax
import jax.numpy as jnp
from jax import lax
from jax.experimental import pallas as pl
from jax.experimental.pallas import tpu as pltpu
from jax.experimental.pallas import tpu_sc as plsc

_N = 10000
_NP = 10240
_H = 128
_E = 320000
_EL = 100000
_EB = 128
_NBAT = _E // _EB
_NW = 32
_RPS = _NP // 16
_ELP = 102400
_NBL = _ELP // _EB

_f32 = jnp.float32


def _mesh():
    return plsc.VectorSubcoreMesh(core_axis_name="c", subcore_axis_name="s")


def _segsum(y, src1d, dst1d):
    _TB = (_NBAT + _NW - 1) // _NW

    @functools.partial(
        pl.kernel,
        out_type=jax.ShapeDtypeStruct((2 * _NP, _H), _f32),
        mesh=_mesh(),
        scratch_types=[
            pltpu.VMEM((2, _EB), jnp.int32),
            pltpu.VMEM((2, _EB), jnp.int32),
            pltpu.VMEM((2, _EB, _H), _f32),
            pltpu.VMEM_SHARED((_NP, _H), _f32),
            pltpu.SemaphoreType.DMA,
            pltpu.SemaphoreType.DMA,
            pltpu.SemaphoreType.DMA,
            pltpu.SemaphoreType.DMA,
        ],
    )
    def k(y_hbm, src_hbm, dst_hbm, s_hbm,
          srcv, dstv, rows, acc, sem_i, sem_g0, sem_g1, sem_s):
        sem_g = (sem_g0, sem_g1)
        c = lax.axis_index("c")
        s = lax.axis_index("s")
        w = s * 2 + c
        r0 = pl.multiple_of(s * _RPS, _RPS)

        def vld(t):
            return (w + t * _NW) < _NBAT

        def idx_copies(t, b):
            o = pl.multiple_of((w + t * _NW) * _EB, _EB)
            return (pltpu.make_async_copy(src_hbm.at[pl.ds(o, _EB)],
                                          srcv.at[b], sem_i),
                    pltpu.make_async_copy(dst_hbm.at[pl.ds(o, _EB)],
                                          dstv.at[b], sem_i))

        def gat_copy(b):
            return pltpu.make_async_copy(y_hbm.at[srcv.at[b]], rows.at[b],
                                         sem_g[b])

        def sct_copy(b):
            return pltpu.make_async_copy(rows.at[b], acc.at[dstv.at[b]],
                                         sem_s)

        @pl.loop(0, _EB)
        def _(i):
            @pl.loop(0, _H // 16)
            def _(j):
                rows[0, i, pl.ds(j * 16, 16)] = jnp.zeros((16,), _f32)

        @pl.loop(0, _RPS // _EB)
        def _(i):
            pltpu.sync_copy(rows.at[0], acc.at[pl.ds(r0 + i * _EB, _EB)])

        plsc.subcore_barrier()

        for cp in idx_copies(0, 0):
            cp.start()
        for cp in idx_copies(1, 1):
            cp.start()

        @pl.loop(0, _TB + 2, step=2)
        def _(T):
            for b in (0, 1):
                t = T + b

                @pl.when((t < _TB) & vld(t))
                def _():
                    for cp in idx_copies(t, b):
                        cp.wait()

                @pl.when((t >= 2) & vld(t - 2))
                def _():
                    sct_copy(b).wait()

                @pl.when((t < _TB) & vld(t))
                def _():
                    gat_copy(b).start()

                @pl.when((t >= 1) & (t - 1 < _TB) & vld(t - 1))
                def _():
                    gat_copy(1 - b).wait()
                    sct_copy(1 - b).start(add=True)

                    @pl.when((t + 1 < _TB) & vld(t + 1))
                    def _():
                        for cp in idx_copies(t + 1, 1 - b):
                            cp.start()

        plsc.subcore_barrier()
        o0 = pl.multiple_of(c * _NP + r0, _RPS)
        pltpu.sync_copy(acc.at[pl.ds(r0, _RPS)], s_hbm.at[pl.ds(o0, _RPS)])

    return k(y, src1d, dst1d)


_HR = _NP // _H
_HRT = _HR // 16


def _deg_hist(dst1d):
    @functools.partial(
        pl.kernel,
        out_type=jax.ShapeDtypeStruct((2 * _NP * 16,), _f32),
        mesh=_mesh(),
        compiler_params=dataclasses.replace(
            pltpu.CompilerParams(), needs_layout_passes=False),
        scratch_types=[
            pltpu.VMEM((2, _EB), jnp.int32),
            pltpu.VMEM((_HR, _H), _f32),
            pltpu.VMEM((_HR,), jnp.int32),
            pltpu.VMEM((_HRT, _H), _f32),
            pltpu.VMEM((_RPS * 16,), _f32),
            pltpu.VMEM_SHARED((_HR, _H), _f32),
            pltpu.SemaphoreType.DMA,
        ],
    )
    def k(dst_hbm, d_hbm, dstv, hist, idv, dbuf, bbuf, sdeg, sem_i):
        c = lax.axis_index("c")
        s = lax.axis_index("s")
        w = s * 2 + c

        def vld(t):
            return (w + t * _NW) < _NBAT

        def idx_copy(t, b):
            o = pl.multiple_of((w + t * _NW) * _EB, _EB)
            return pltpu.make_async_copy(dst_hbm.at[pl.ds(o, _EB)],
                                         dstv.at[b], sem_i)

        @pl.loop(0, _HR)
        def _(i):
            @pl.loop(0, _H // 16)
            def _(j):
                hist[i, pl.ds(j * 16, 16)] = jnp.zeros((16,), _f32)

        @pl.loop(0, _HR // 16)
        def _(i):
            idv[pl.ds(i * 16, 16)] = lax.iota(jnp.int32, 16) + i * 16

        @pl.when(s == 0)
        def _():
            pltpu.sync_copy(hist, sdeg)

        plsc.subcore_barrier()

        ones16 = jnp.ones((16,), _f32)

        idx_copy(0, 0).start()
        idx_copy(1, 1).start()

        @pl.loop(0, (_NBAT + _NW - 1) // _NW + 1, step=2)
        def _(T):
            for b in (0, 1):
                t = T + b

                @pl.when(vld(t))
                def _():
                    idx_copy(t, b).wait()

                    @pl.loop(0, _EB // 16)
                    def _(kk):
                        d16 = dstv[b, pl.ds(kk * 16, 16)]
                        r = lax.shift_right_logical(d16, 7)
                        cl = lax.bitwise_and(d16, 127)
                        plsc.addupdate_scatter(hist, [r, cl], ones16)

                    @pl.when(vld(t + 2))
                    def _():
                        idx_copy(t + 2, b).start()

        pltpu.sync_copy(hist, sdeg.at[idv], add=True)
        plsc.subcore_barrier()

        pltpu.sync_copy(sdeg.at[pl.ds(s * _HRT, _HRT)], dbuf)

        @pl.loop(0, _HRT)
        def _(r):
            @pl.loop(0, _H // 16)
            def _(lg):
                v = dbuf[r, pl.ds(lg * 16, 16)]
                base = (r * _H + lg * 16) * 16
                for ll in range(16):
                    bbuf[pl.ds(base + ll * 16, 16)] = jnp.full((16,), v[ll], _f32)

        o0 = pl.multiple_of((c * _NP + s * _RPS) * 16, 8)
        pltpu.sync_copy(bbuf, d_hbm.at[pl.ds(o0, _RPS * 16)])

    return k(dst1d)


def _edge_decode(u, v, e0, e1, wd2f, bd2p):
    _TB = _NBL // _NW

    @functools.partial(
        pl.kernel,
        out_type=jax.ShapeDtypeStruct((_ELP,), _f32),
        mesh=_mesh(),
        compiler_params=dataclasses.replace(
            pltpu.CompilerParams(), needs_layout_passes=False),
        scratch_types=[
            pltpu.VMEM((2, _EB), jnp.int32),
            pltpu.VMEM((2, _EB), jnp.int32),
            pltpu.VMEM((2, _EB, _H), _f32),
            pltpu.VMEM((2, _EB, _H), _f32),
            pltpu.VMEM((2, _EB), _f32),
            pltpu.VMEM((_H,), _f32),
            pltpu.VMEM((16,), _f32),
            pltpu.VMEM((16, 16), _f32),
            pltpu.SemaphoreType.DMA,
            pltpu.SemaphoreType.DMA,
            pltpu.SemaphoreType.DMA,
            pltpu.SemaphoreType.DMA,
        ],
    )
    def k(u_hbm, v_hbm, e0_hbm, e1_hbm, w_hbm, b_hbm, o_hbm,
          i0, i1, ru, rv, obuf, wv, bv, macc,
          sem_i, sem_g0, sem_g1, sem_w):
        sem_g = (sem_g0, sem_g1)
        c = lax.axis_index("c")
        s = lax.axis_index("s")
        w = s * 2 + c

        pltpu.sync_copy(w_hbm, wv)
        pltpu.sync_copy(b_hbm, bv)
        wch = [wv[pl.ds(kk * 16, 16)] for kk in range(_H // 16)]
        bd2s = bv[...][0]
        iota16 = lax.iota(jnp.int32, 16)

        def boff(t):
            return pl.multiple_of((w + t * _NW) * _EB, _EB)

        def idx_copies(t, b):
            o = boff(t)
            return (pltpu.make_async_copy(e0_hbm.at[pl.ds(o, _EB)],
                                          i0.at[b], sem_i),
                    pltpu.make_async_copy(e1_hbm.at[pl.ds(o, _EB)],
                                          i1.at[b], sem_i))

        def gat_copies(b):
            return (pltpu.make_async_copy(u_hbm.at[i0.at[b]], ru.at[b],
                                          sem_g[b]),
                    pltpu.make_async_copy(v_hbm.at[i1.at[b]], rv.at[b],
                                          sem_g[b]))

        def wr_copy(t, b):
            return pltpu.make_async_copy(obuf.at[b],
                                         o_hbm.at[pl.ds(boff(t), _EB)], sem_w)

        def compute(b):
            @pl.loop(0, _EB // 16)
            def _(g):
                for i in range(16):
                    r = g * 16 + i
                    acc = jnp.zeros((16,), _f32)
                    for kk in range(_H // 16):
                        uu = ru[b, r, pl.ds(kk * 16, 16)]
                        vv = rv[b, r, pl.ds(kk * 16, 16)]
                        acc = acc + jnp.maximum(uu + vv, 0.0) * wch[kk]
                    macc[i, :] = acc

                o = jnp.full((16,), bd2s, _f32)
                for l in range(16):
                    o = o + plsc.load_gather(
                        macc, [iota16, jnp.full((16,), l, jnp.int32)])
                obuf[b, pl.ds(g * 16, 16)] = o

        for cp in idx_copies(0, 0):
            cp.start()
        for cp in idx_copies(1, 1):
            cp.start()

        @pl.loop(0, _TB + 1, step=2)
        def _(T):
            for b in (0, 1):
                t = T + b

                @pl.when(t < _TB)
                def _():
                    for cp in idx_copies(t, b):
                        cp.wait()

                @pl.when(t >= 2)
                def _():
                    wr_copy(t - 2, b).wait()

                @pl.when(t < _TB)
                def _():
                    for cp in gat_copies(b):
                        cp.start()

                @pl.when(t >= 1)
                def _():
                    for cp in gat_copies(1 - b):
                        cp.wait()
                    compute(1 - b)
                    wr_copy(t - 1, 1 - b).start()

                    @pl.when(t + 1 < _TB)
                    def _():
                        for cp in idx_copies(t + 1, 1 - b):
                            cp.start()

        wr_copy(_TB - 1, (_TB - 1) % 2).wait()

    return k(u, v, e0, e1, wd2f, bd2p)


def _tc_pre(x, W1l, W1r, b1):
    R = 1024

    def body(x_ref, wl_ref, wr_ref, b_ref, y_ref, xr_ref):
        xx = x_ref[...]
        y_ref[...] = jnp.dot(xx, wl_ref[...], preferred_element_type=_f32)
        xr_ref[...] = (jnp.dot(xx, wr_ref[...], preferred_element_type=_f32)
                       + b_ref[...])

    return pl.pallas_call(
        body,
        grid=(_NP // R,),
        in_specs=[pl.BlockSpec((R, _H), lambda i: (i, 0)),
                  pl.BlockSpec((_H, _H), lambda i: (0, 0)),
                  pl.BlockSpec((_H, _H), lambda i: (0, 0)),
                  pl.BlockSpec((1, _H), lambda i: (0, 0))],
        out_specs=[pl.BlockSpec((R, _H), lambda i: (i, 0)),
                   pl.BlockSpec((R, _H), lambda i: (i, 0))],
        out_shape=[jax.ShapeDtypeStruct((_NP, _H), _f32)] * 2,
    )(x, W1l, W1r, b1.reshape(1, _H))


def _tc_mid(s1p, degp, xr1b, W2l, W2r, b2):
    R = 1024

    def body(sp, dp, xr, wl, wr, b, y2, hr):
        ssum = sp[0] + sp[1]
        dg = dp[0, :, 0:1] + dp[1, :, 0:1]
        inv = 1.0 / jnp.maximum(dg, 1.0)
        h = jnp.maximum(ssum * inv + xr[...], 0.0)
        y2[...] = jnp.dot(h, wl[...], preferred_element_type=_f32)
        hr[...] = jnp.dot(h, wr[...], preferred_element_type=_f32) + b[...]

    return pl.pallas_call(
        body,
        grid=(_NP // R,),
        in_specs=[pl.BlockSpec((2, R, _H), lambda i: (0, i, 0)),
                  pl.BlockSpec((2, R, 16), lambda i: (0, i, 0)),
                  pl.BlockSpec((R, _H), lambda i: (i, 0)),
                  pl.BlockSpec((_H, _H), lambda i: (0, 0)),
                  pl.BlockSpec((_H, _H), lambda i: (0, 0)),
                  pl.BlockSpec((1, _H), lambda i: (0, 0))],
        out_specs=[pl.BlockSpec((R, _H), lambda i: (i, 0)),
                   pl.BlockSpec((R, _H), lambda i: (i, 0))],
        out_shape=[jax.ShapeDtypeStruct((_NP, _H), _f32)] * 2,
    )(s1p, degp, xr1b, W2l, W2r, b2.reshape(1, _H))


def _tc_z(s2p, degp, hr2b, Wd1, bd1):
    R = 1024

    def body(sp, dp, hr, wd, b, u, v):
        ssum = sp[0] + sp[1]
        dg = dp[0, :, 0:1] + dp[1, :, 0:1]
        inv = 1.0 / jnp.maximum(dg, 1.0)
        z = ssum * inv + hr[...]
        wa = wd[0:_H, :]
        wb = wd[_H:2 * _H, :]
        u[...] = jnp.dot(z, wa, preferred_element_type=_f32) + b[...]
        v[...] = jnp.dot(z, wb, preferred_element_type=_f32)

    return pl.pallas_call(
        body,
        grid=(_NP // R,),
        in_specs=[pl.BlockSpec((2, R, _H), lambda i: (0, i, 0)),
                  pl.BlockSpec((2, R, 16), lambda i: (0, i, 0)),
                  pl.BlockSpec((R, _H), lambda i: (i, 0)),
                  pl.BlockSpec((2 * _H, _H), lambda i: (0, 0)),
                  pl.BlockSpec((1, _H), lambda i: (0, 0))],
        out_specs=[pl.BlockSpec((R, _H), lambda i: (i, 0)),
                   pl.BlockSpec((R, _H), lambda i: (i, 0))],
        out_shape=[jax.ShapeDtypeStruct((_NP, _H), _f32)] * 2,
    )(s2p, degp, hr2b, Wd1, bd1.reshape(1, _H))


def kernel(x, edge_index, edge_label_index,
           W1l, b1, W1r, W2l, b2, W2r, Wd1, bd1, Wd2, bd2):
    xp = jnp.pad(x, ((0, _NP - _N), (0, 0)))
    src1d = edge_index[0]
    dst1d = edge_index[1]

    y1, xr1b = _tc_pre(xp, W1l, W1r, b1)
    degf = _deg_hist(dst1d)
    s1f = _segsum(y1, src1d, dst1d)
    s1p = s1f.reshape(2, _NP, _H)
    degp = degf.reshape(2, _NP, 16)

    y2, hr2b = _tc_mid(s1p, degp, xr1b, W2l, W2r, b2)
    s2f = _segsum(y2, src1d, dst1d)
    s2p = s2f.reshape(2, _NP, _H)

    u, v = _tc_z(s2p, degp, hr2b, Wd1, bd1)

    pad = _ELP - _EL
    e0 = jnp.pad(edge_label_index[0], (0, pad))
    e1 = jnp.pad(edge_label_index[1], (0, pad))
    out = _edge_decode(u, v, e0, e1, Wd2.reshape(_H),
                       jnp.pad(bd2, (0, 15)))
    return out[:_EL]


# speedup vs baseline: 9.0619x; 1.0042x over previous
import dataclasses
import functools

import jax
import jax.numpy as jnp
from jax import lax
from jax.experimental import pallas as pl
from jax.experimental.pallas import tpu as pltpu
from jax.experimental.pallas import tpu_sc as plsc

_N = 10000
_NP = 10240
_H = 128
_E = 320000
_EL = 100000
_EB = 128
_NBAT = _E // _EB
_NW = 32
_RPS = _NP // 16
_ELP = 102400
_NBL = _ELP // _EB

_f32 = jnp.float32


def _mesh():
    return plsc.VectorSubcoreMesh(core_axis_name="c", subcore_axis_name="s")


def _segsum(y, src1d, dst1d):
    _TB = (_NBAT + _NW - 1) // _NW

    @functools.partial(
        pl.kernel,
        out_type=jax.ShapeDtypeStruct((2 * _NP, _H), _f32),
        mesh=_mesh(),
        scratch_types=[
            pltpu.VMEM((2, _EB), jnp.int32),
            pltpu.VMEM((2, _EB), jnp.int32),
            pltpu.VMEM((2, _EB, _H), _f32),
            pltpu.VMEM_SHARED((_NP, _H), _f32),
            pltpu.SemaphoreType.DMA,
            pltpu.SemaphoreType.DMA,
            pltpu.SemaphoreType.DMA,
            pltpu.SemaphoreType.DMA,
        ],
    )
    def k(y_hbm, src_hbm, dst_hbm, s_hbm,
          srcv, dstv, rows, acc, sem_i, sem_g0, sem_g1, sem_s):
        sem_g = (sem_g0, sem_g1)
        c = lax.axis_index("c")
        s = lax.axis_index("s")
        w = s * 2 + c
        r0 = pl.multiple_of(s * _RPS, _RPS)

        def vld(t):
            return (w + t * _NW) < _NBAT

        def idx_copies(t, b):
            o = pl.multiple_of((w + t * _NW) * _EB, _EB)
            return (pltpu.make_async_copy(src_hbm.at[pl.ds(o, _EB)],
                                          srcv.at[b], sem_i),
                    pltpu.make_async_copy(dst_hbm.at[pl.ds(o, _EB)],
                                          dstv.at[b], sem_i))

        def gat_copy(b):
            return pltpu.make_async_copy(y_hbm.at[srcv.at[b]], rows.at[b],
                                         sem_g[b])

        def sct_copy(b):
            return pltpu.make_async_copy(rows.at[b], acc.at[dstv.at[b]],
                                         sem_s)

        @pl.loop(0, _EB)
        def _(i):
            @pl.loop(0, _H // 16)
            def _(j):
                rows[0, i, pl.ds(j * 16, 16)] = jnp.zeros((16,), _f32)

        @pl.loop(0, _RPS // _EB)
        def _(i):
            pltpu.sync_copy(rows.at[0], acc.at[pl.ds(r0 + i * _EB, _EB)])

        plsc.subcore_barrier()

        for cp in idx_copies(0, 0):
            cp.start()
        for cp in idx_copies(1, 1):
            cp.start()

        @pl.loop(0, _TB + 2, step=2)
        def _(T):
            for b in (0, 1):
                t = T + b

                @pl.when((t < _TB) & vld(t))
                def _():
                    for cp in idx_copies(t, b):
                        cp.wait()

                @pl.when((t >= 2) & vld(t - 2))
                def _():
                    sct_copy(b).wait()

                @pl.when((t < _TB) & vld(t))
                def _():
                    gat_copy(b).start()

                @pl.when((t >= 1) & (t - 1 < _TB) & vld(t - 1))
                def _():
                    gat_copy(1 - b).wait()
                    sct_copy(1 - b).start(add=True)

                    @pl.when((t + 1 < _TB) & vld(t + 1))
                    def _():
                        for cp in idx_copies(t + 1, 1 - b):
                            cp.start()

        plsc.subcore_barrier()
        o0 = pl.multiple_of(c * _NP + r0, _RPS)
        pltpu.sync_copy(acc.at[pl.ds(r0, _RPS)], s_hbm.at[pl.ds(o0, _RPS)])

    return k(y, src1d, dst1d)


_HR = _NP // _H
_HRT = _HR // 16


def _deg_hist(dst1d):
    @functools.partial(
        pl.kernel,
        out_type=jax.ShapeDtypeStruct((2 * _NP * 16,), _f32),
        mesh=_mesh(),
        compiler_params=dataclasses.replace(
            pltpu.CompilerParams(), needs_layout_passes=False),
        scratch_types=[
            pltpu.VMEM((2, _EB), jnp.int32),
            pltpu.VMEM((_HR, _H), _f32),
            pltpu.VMEM((_HR,), jnp.int32),
            pltpu.VMEM((_HRT, _H), _f32),
            pltpu.VMEM((_RPS * 16,), _f32),
            pltpu.VMEM_SHARED((_HR, _H), _f32),
            pltpu.SemaphoreType.DMA,
        ],
    )
    def k(dst_hbm, d_hbm, dstv, hist, idv, dbuf, bbuf, sdeg, sem_i):
        c = lax.axis_index("c")
        s = lax.axis_index("s")
        w = s * 2 + c

        def vld(t):
            return (w + t * _NW) < _NBAT

        def idx_copy(t, b):
            o = pl.multiple_of((w + t * _NW) * _EB, _EB)
            return pltpu.make_async_copy(dst_hbm.at[pl.ds(o, _EB)],
                                         dstv.at[b], sem_i)

        @pl.loop(0, _HR)
        def _(i):
            @pl.loop(0, _H // 16)
            def _(j):
                hist[i, pl.ds(j * 16, 16)] = jnp.zeros((16,), _f32)

        @pl.loop(0, _HR // 16)
        def _(i):
            idv[pl.ds(i * 16, 16)] = lax.iota(jnp.int32, 16) + i * 16

        @pl.when(s == 0)
        def _():
            pltpu.sync_copy(hist, sdeg)

        plsc.subcore_barrier()

        ones16 = jnp.ones((16,), _f32)

        idx_copy(0, 0).start()
        idx_copy(1, 1).start()

        @pl.loop(0, (_NBAT + _NW - 1) // _NW + 1, step=2)
        def _(T):
            for b in (0, 1):
                t = T + b

                @pl.when(vld(t))
                def _():
                    idx_copy(t, b).wait()

                    @pl.loop(0, _EB // 16)
                    def _(kk):
                        d16 = dstv[b, pl.ds(kk * 16, 16)]
                        r = lax.shift_right_logical(d16, 7)
                        cl = lax.bitwise_and(d16, 127)
                        plsc.addupdate_scatter(hist, [r, cl], ones16)

                    @pl.when(vld(t + 2))
                    def _():
                        idx_copy(t + 2, b).start()

        pltpu.sync_copy(hist, sdeg.at[idv], add=True)
        plsc.subcore_barrier()

        pltpu.sync_copy(sdeg.at[pl.ds(s * _HRT, _HRT)], dbuf)

        @pl.loop(0, _HRT)
        def _(r):
            @pl.loop(0, _H // 16)
            def _(lg):
                v = dbuf[r, pl.ds(lg * 16, 16)]
                base = (r * _H + lg * 16) * 16
                for ll in range(16):
                    bbuf[pl.ds(base + ll * 16, 16)] = jnp.full((16,), v[ll], _f32)

        o0 = pl.multiple_of((c * _NP + s * _RPS) * 16, 8)
        pltpu.sync_copy(bbuf, d_hbm.at[pl.ds(o0, _RPS * 16)])

    return k(dst1d)


def _edge_decode(u, v, e0, e1, wd2f, bd2p):
    _TB = _NBL // _NW

    @functools.partial(
        pl.kernel,
        out_type=jax.ShapeDtypeStruct((_ELP,), _f32),
        mesh=_mesh(),
        compiler_params=dataclasses.replace(
            pltpu.CompilerParams(), needs_layout_passes=False),
        scratch_types=[
            pltpu.VMEM((3, _EB), jnp.int32),
            pltpu.VMEM((3, _EB), jnp.int32),
            pltpu.VMEM((3, _EB, _H), _f32),
            pltpu.VMEM((3, _EB, _H), _f32),
            pltpu.VMEM((3, _EB), _f32),
            pltpu.VMEM((_H,), _f32),
            pltpu.VMEM((16,), _f32),
            pltpu.VMEM((16, 16), _f32),
            pltpu.SemaphoreType.DMA,
            pltpu.SemaphoreType.DMA,
            pltpu.SemaphoreType.DMA,
            pltpu.SemaphoreType.DMA,
            pltpu.SemaphoreType.DMA,
        ],
    )
    def k(u_hbm, v_hbm, e0_hbm, e1_hbm, w_hbm, b_hbm, o_hbm,
          i0, i1, ru, rv, obuf, wv, bv, macc,
          sem_i, sem_g0, sem_g1, sem_g2, sem_w):
        sem_g = (sem_g0, sem_g1, sem_g2)
        c = lax.axis_index("c")
        s = lax.axis_index("s")
        w = s * 2 + c

        pltpu.sync_copy(w_hbm, wv)
        pltpu.sync_copy(b_hbm, bv)
        wch = [wv[pl.ds(kk * 16, 16)] for kk in range(_H // 16)]
        bd2s = bv[...][0]
        iota16 = lax.iota(jnp.int32, 16)

        def boff(t):
            return pl.multiple_of((w + t * _NW) * _EB, _EB)

        def idx_copies(t, b):
            o = boff(t)
            return (pltpu.make_async_copy(e0_hbm.at[pl.ds(o, _EB)],
                                          i0.at[b], sem_i),
                    pltpu.make_async_copy(e1_hbm.at[pl.ds(o, _EB)],
                                          i1.at[b], sem_i))

        def gat_copies(b):
            return (pltpu.make_async_copy(u_hbm.at[i0.at[b]], ru.at[b],
                                          sem_g[b]),
                    pltpu.make_async_copy(v_hbm.at[i1.at[b]], rv.at[b],
                                          sem_g[b]))

        def wr_copy(t, b):
            return pltpu.make_async_copy(obuf.at[b],
                                         o_hbm.at[pl.ds(boff(t), _EB)], sem_w)

        def compute(b):
            @pl.loop(0, _EB // 16)
            def _(g):
                for i in range(16):
                    r = g * 16 + i
                    acc = jnp.zeros((16,), _f32)
                    for kk in range(_H // 16):
                        uu = ru[b, r, pl.ds(kk * 16, 16)]
                        vv = rv[b, r, pl.ds(kk * 16, 16)]
                        acc = acc + jnp.maximum(uu + vv, 0.0) * wch[kk]
                    macc[i, :] = acc

                o = jnp.full((16,), bd2s, _f32)
                for l in range(16):
                    o = o + plsc.load_gather(
                        macc, [iota16, jnp.full((16,), l, jnp.int32)])
                obuf[b, pl.ds(g * 16, 16)] = o

        for pt in (0, 1, 2):
            for cp in idx_copies(pt, pt):
                cp.start()

        @pl.loop(0, _TB + 2, step=3)
        def _(T):
            for b in (0, 1, 2):
                t = T + b
                bm2 = (b + 1) % 3

                @pl.when(t < _TB)
                def _():
                    for cp in idx_copies(t, b):
                        cp.wait()

                @pl.when(t >= 3)
                def _():
                    wr_copy(t - 3, b).wait()

                @pl.when(t < _TB)
                def _():
                    for cp in gat_copies(b):
                        cp.start()

                @pl.when(t >= 2)
                def _():
                    for cp in gat_copies(bm2):
                        cp.wait()
                    compute(bm2)
                    wr_copy(t - 2, bm2).start()

                    @pl.when(t + 1 < _TB)
                    def _():
                        for cp in idx_copies(t + 1, bm2):
                            cp.start()

        wr_copy(_TB - 1, (_TB - 1) % 3).wait()

    return k(u, v, e0, e1, wd2f, bd2p)


def _tc_pre(x, W1l, W1r, b1):
    R = 1024

    def body(x_ref, wl_ref, wr_ref, b_ref, y_ref, xr_ref):
        xx = x_ref[...]
        y_ref[...] = jnp.dot(xx, wl_ref[...], preferred_element_type=_f32)
        xr_ref[...] = (jnp.dot(xx, wr_ref[...], preferred_element_type=_f32)
                       + b_ref[...])

    return pl.pallas_call(
        body,
        grid=(_NP // R,),
        in_specs=[pl.BlockSpec((R, _H), lambda i: (i, 0)),
                  pl.BlockSpec((_H, _H), lambda i: (0, 0)),
                  pl.BlockSpec((_H, _H), lambda i: (0, 0)),
                  pl.BlockSpec((1, _H), lambda i: (0, 0))],
        out_specs=[pl.BlockSpec((R, _H), lambda i: (i, 0)),
                   pl.BlockSpec((R, _H), lambda i: (i, 0))],
        out_shape=[jax.ShapeDtypeStruct((_NP, _H), _f32)] * 2,
    )(x, W1l, W1r, b1.reshape(1, _H))


def _tc_mid(s1p, degp, xr1b, W2l, W2r, b2):
    R = 1024

    def body(sp, dp, xr, wl, wr, b, y2, hr):
        ssum = sp[0] + sp[1]
        dg = dp[0, :, 0:1] + dp[1, :, 0:1]
        inv = 1.0 / jnp.maximum(dg, 1.0)
        h = jnp.maximum(ssum * inv + xr[...], 0.0)
        y2[...] = jnp.dot(h, wl[...], preferred_element_type=_f32)
        hr[...] = jnp.dot(h, wr[...], preferred_element_type=_f32) + b[...]

    return pl.pallas_call(
        body,
        grid=(_NP // R,),
        in_specs=[pl.BlockSpec((2, R, _H), lambda i: (0, i, 0)),
                  pl.BlockSpec((2, R, 16), lambda i: (0, i, 0)),
                  pl.BlockSpec((R, _H), lambda i: (i, 0)),
                  pl.BlockSpec((_H, _H), lambda i: (0, 0)),
                  pl.BlockSpec((_H, _H), lambda i: (0, 0)),
                  pl.BlockSpec((1, _H), lambda i: (0, 0))],
        out_specs=[pl.BlockSpec((R, _H), lambda i: (i, 0)),
                   pl.BlockSpec((R, _H), lambda i: (i, 0))],
        out_shape=[jax.ShapeDtypeStruct((_NP, _H), _f32)] * 2,
    )(s1p, degp, xr1b, W2l, W2r, b2.reshape(1, _H))


def _tc_z(s2p, degp, hr2b, Wd1, bd1):
    R = 1024

    def body(sp, dp, hr, wd, b, u, v):
        ssum = sp[0] + sp[1]
        dg = dp[0, :, 0:1] + dp[1, :, 0:1]
        inv = 1.0 / jnp.maximum(dg, 1.0)
        z = ssum * inv + hr[...]
        wa = wd[0:_H, :]
        wb = wd[_H:2 * _H, :]
        u[...] = jnp.dot(z, wa, preferred_element_type=_f32) + b[...]
        v[...] = jnp.dot(z, wb, preferred_element_type=_f32)

    return pl.pallas_call(
        body,
        grid=(_NP // R,),
        in_specs=[pl.BlockSpec((2, R, _H), lambda i: (0, i, 0)),
                  pl.BlockSpec((2, R, 16), lambda i: (0, i, 0)),
                  pl.BlockSpec((R, _H), lambda i: (i, 0)),
                  pl.BlockSpec((2 * _H, _H), lambda i: (0, 0)),
                  pl.BlockSpec((1, _H), lambda i: (0, 0))],
        out_specs=[pl.BlockSpec((R, _H), lambda i: (i, 0)),
                   pl.BlockSpec((R, _H), lambda i: (i, 0))],
        out_shape=[jax.ShapeDtypeStruct((_NP, _H), _f32)] * 2,
    )(s2p, degp, hr2b, Wd1, bd1.reshape(1, _H))


def kernel(x, edge_index, edge_label_index,
           W1l, b1, W1r, W2l, b2, W2r, Wd1, bd1, Wd2, bd2):
    xp = jnp.pad(x, ((0, _NP - _N), (0, 0)))
    src1d = edge_index[0]
    dst1d = edge_index[1]

    y1, xr1b = _tc_pre(xp, W1l, W1r, b1)
    degf = _deg_hist(dst1d)
    s1f = _segsum(y1, src1d, dst1d)
    s1p = s1f.reshape(2, _NP, _H)
    degp = degf.reshape(2, _NP, 16)

    y2, hr2b = _tc_mid(s1p, degp, xr1b, W2l, W2r, b2)
    s2f = _segsum(y2, src1d, dst1d)
    s2p = s2f.reshape(2, _NP, _H)

    u, v = _tc_z(s2p, degp, hr2b, Wd1, bd1)

    pad = _ELP - _EL
    e0 = jnp.pad(edge_label_index[0], (0, pad))
    e1 = jnp.pad(edge_label_index[1], (0, pad))
    out = _edge_decode(u, v, e0, e1, Wd2.reshape(_H),
                       jnp.pad(bd2, (0, 15)))
    return out[:_EL]
